# baseline (speedup 1.0000x reference)
import functools

import jax
import jax.numpy as jnp
from jax import lax
from jax.experimental import pallas as pl
from jax.experimental.pallas import tpu as pltpu
from jax.experimental.pallas import tpu_sc as plsc

LN = 8
C = 80
NC = 2
NS = 16
NW = NC * NS


def _sc_aggregate(N, D, E, x, e3, ones, zsum, zcnt):
    n_chunks = E // C
    chunks_pw = n_chunks // NW
    rpt = N // NS

    mesh = plsc.VectorSubcoreMesh(core_axis_name="core",
                                  subcore_axis_name="subcore")

    @functools.partial(
        pl.kernel,
        out_type=[
            jax.ShapeDtypeStruct((NC * N, D), jnp.float32),
            jax.ShapeDtypeStruct((NC * N, LN), jnp.float32),
        ],
        mesh=mesh,
        scratch_types=[
            pltpu.VMEM((2, C), jnp.int32),
            pltpu.VMEM((2, C), jnp.int32),
            pltpu.VMEM((2, C), jnp.int32),
            pltpu.VMEM((2, C), jnp.int32),
            pltpu.VMEM((C, D), jnp.float32),
            pltpu.VMEM((C, D), jnp.float32),
            pltpu.VMEM((C, D), jnp.float32),
            pltpu.VMEM((C, D), jnp.float32),
            pltpu.VMEM((C, LN), jnp.float32),
            pltpu.VMEM_SHARED((N, D), jnp.float32),
            pltpu.VMEM_SHARED((N, LN), jnp.float32),
            pltpu.SemaphoreType.DMA,
            pltpu.SemaphoreType.DMA,
            pltpu.SemaphoreType.DMA,
            pltpu.SemaphoreType.DMA,
            pltpu.SemaphoreType.DMA,
            pltpu.SemaphoreType.DMA,
            pltpu.SemaphoreType.DMA,
            pltpu.SemaphoreType.DMA,
            pltpu.SemaphoreType.DMA,
        ],
        compiler_params=pltpu.CompilerParams(use_tc_tiling_on_sc=False),
    )
    def sc_kernel(x_hbm, e3_hbm, ones_hbm, zsum_hbm, zcnt_hbm,
                  out_sum, out_cnt,
                  ib0, ib1, ib2, ib3, rb0, rb1, rb2, rb3, ones_v, sum_sh,
                  cnt_sh, si0, si1, si2, si3, sg0, sg1, sg2, sg3, sem_c):
        c = lax.axis_index("core")
        s = lax.axis_index("subcore")
        w = c * NS + s
        cbase = w * chunks_pw

        ibuf = (ib0, ib1, ib2, ib3)
        bufs = (rb0, rb1, rb2, rb3)
        sem_i = (si0, si1, si2, si3)
        sems = (sg0, sg1, sg2, sg3)

        pltpu.sync_copy(zsum_hbm.at[pl.ds(s * rpt, rpt)],
                        sum_sh.at[pl.ds(s * rpt, rpt)])
        pltpu.sync_copy(zcnt_hbm.at[pl.ds(s * rpt, rpt)],
                        cnt_sh.at[pl.ds(s * rpt, rpt)])
        pltpu.sync_copy(ones_hbm, ones_v)
        plsc.subcore_barrier()

        def load_idx(k, m):
            pltpu.async_copy(e3_hbm.at[cbase + k], ibuf[m], sem_i[m])

        def wait_idx(m):
            pltpu.make_async_copy(e3_hbm.at[0], ibuf[m], sem_i[m]).wait()

        def start_gather(m):
            pltpu.async_copy(x_hbm.at[ibuf[m].at[0]], bufs[m], sems[m])

        def wait_gather(m):
            pltpu.make_async_copy(x_hbm.at[ibuf[m].at[0]], bufs[m],
                                  sems[m]).wait()

        def scatter(b):
            pltpu.async_copy(ones_v, cnt_sh.at[ibuf[b].at[1]], sem_c,
                             add=True)
            pltpu.sync_copy(bufs[b], sum_sh.at[ibuf[b].at[1]], add=True)

        def wait_cnt():
            pltpu.make_async_copy(ones_v, cnt_sh.at[ibuf[0].at[1]],
                                  sem_c).wait()

        for m in range(4):
            load_idx(m, m)
        for m in range(3):
            wait_idx(m)
            start_gather(m)

        for k in (0, 1):
            b = k % 4
            wait_gather(b)
            scatter(b)
            load_idx(k + 4, b)
            wait_idx((b + 3) % 4)
            start_gather((b + 3) % 4)

        def slot(k, b):
            wait_gather(b)
            wait_cnt()
            scatter(b)
            load_idx(k + 4, b)
            wait_idx((b + 3) % 4)
            start_gather((b + 3) % 4)

        n_quad = (chunks_pw - 7) // 4

        @pl.loop(0, n_quad)
        def _(q):
            k = 4 * q + 2
            slot(k, 2)
            slot(k + 1, 3)
            slot(k + 2, 0)
            slot(k + 3, 1)

        for k in range(4 * n_quad + 2, chunks_pw):
            b = k % 4
            wait_gather(b)
            wait_cnt()
            scatter(b)
            if k + 4 < chunks_pw:
                load_idx(k + 4, b)
            if k + 3 < chunks_pw:
                wait_idx((b + 3) % 4)
                start_gather((b + 3) % 4)

        wait_cnt()
        wait_cnt()

        plsc.subcore_barrier()
        base = c * N + s * rpt
        pltpu.sync_copy(sum_sh.at[pl.ds(s * rpt, rpt)],
                        out_sum.at[pl.ds(base, rpt)])
        pltpu.sync_copy(cnt_sh.at[pl.ds(s * rpt, rpt)],
                        out_cnt.at[pl.ds(base, rpt)])

    return sc_kernel(x, e3, ones, zsum, zcnt)


def _tc_head(N, D, H, x, psum, pcnt, w1l_t, b1l, w1r_t, w2_t, b2):
    R = 1000
    G = N // R

    def body(x_r, p0_r, p1_r, c0_r, c1_r, w1l_r, b1l_r, w1r_r, w2_r, b2_r,
             o_r):
        ssum = p0_r[...] + p1_r[...]
        cnt = c0_r[...][:, :1] + c1_r[...][:, :1]
        agg = ssum / jnp.maximum(cnt, 1.0)
        h = lax.dot_general(agg, w1l_r[...], (((1,), (0,)), ((), ())),
                            preferred_element_type=jnp.float32)
        h = h + lax.dot_general(x_r[...], w1r_r[...], (((1,), (0,)), ((), ())),
                                preferred_element_type=jnp.float32)
        h = jnp.maximum(h + b1l_r[...], 0.0)
        o = lax.dot_general(h, w2_r[...], (((1,), (0,)), ((), ())),
                            preferred_element_type=jnp.float32)
        o_r[...] = jnp.maximum(o + b2_r[...], 0.0)

    return pl.pallas_call(
        body,
        grid=(G,),
        in_specs=[
            pl.BlockSpec((R, D), lambda i: (i, 0)),
            pl.BlockSpec((R, D), lambda i: (i, 0)),
            pl.BlockSpec((R, D), lambda i: (i + G, 0)),
            pl.BlockSpec((R, LN), lambda i: (i, 0)),
            pl.BlockSpec((R, LN), lambda i: (i + G, 0)),
            pl.BlockSpec((D, D), lambda i: (0, 0)),
            pl.BlockSpec((1, D), lambda i: (0, 0)),
            pl.BlockSpec((D, D), lambda i: (0, 0)),
            pl.BlockSpec((D, H), lambda i: (0, 0)),
            pl.BlockSpec((1, H), lambda i: (0, 0)),
        ],
        out_specs=pl.BlockSpec((R, H), lambda i: (i, 0)),
        out_shape=jax.ShapeDtypeStruct((N, H), jnp.float32),
    )(x, psum, psum, pcnt, pcnt, w1l_t, b1l, w1r_t, w2_t, b2)


def kernel(x, edge_index, W1l, b1l, W1r, W2, b2):
    N, D = x.shape
    E = edge_index.shape[1]
    H = W2.shape[0]
    assert E % (NW * C) == 0 and N % NS == 0
    assert E // (NW * C) >= 3

    e3 = edge_index.reshape(2, E // C, C).transpose(1, 0, 2)
    ones = jnp.ones((C, LN), jnp.float32)
    zsum = jnp.zeros((N, D), jnp.float32)
    zcnt = jnp.zeros((N, LN), jnp.float32)

    psum, pcnt = _sc_aggregate(N, D, E, x, e3, ones, zsum, zcnt)
    return _tc_head(N, D, H, x, psum, pcnt, W1l.T, b1l.reshape(1, D),
                    W1r.T, W2.T, b2.reshape(1, H))


# speedup vs baseline: 1.0738x; 1.0738x over previous
import functools

import jax
import jax.numpy as jnp
from jax import lax
from jax.experimental import pallas as pl
from jax.experimental.pallas import tpu as pltpu
from jax.experimental.pallas import tpu_sc as plsc

LN = 8
C = 80
NC = 2
NS = 16
NW = NC * NS


def _sc_aggregate(N, D, E, x, src1, dst2, ones, zsum, zcnt):
    n_chunks = E // C
    chunks_pw = n_chunks // NW
    rpt = N // NS

    mesh = plsc.VectorSubcoreMesh(core_axis_name="core",
                                  subcore_axis_name="subcore")

    @functools.partial(
        pl.kernel,
        out_type=[
            jax.ShapeDtypeStruct((NC * N, D), jnp.float32),
            jax.ShapeDtypeStruct((NC * N, LN), jnp.float32),
        ],
        mesh=mesh,
        scratch_types=[
            pltpu.VMEM((C,), jnp.int32),
            pltpu.VMEM((C,), jnp.int32),
            pltpu.VMEM((C,), jnp.int32),
            pltpu.VMEM((chunks_pw, C), jnp.int32),
            pltpu.VMEM((C, D), jnp.float32),
            pltpu.VMEM((C, D), jnp.float32),
            pltpu.VMEM((C, D), jnp.float32),
            pltpu.VMEM((C, LN), jnp.float32),
            pltpu.VMEM_SHARED((N, D), jnp.float32),
            pltpu.VMEM_SHARED((N, LN), jnp.float32),
            pltpu.SemaphoreType.DMA,
            pltpu.SemaphoreType.DMA,
            pltpu.SemaphoreType.DMA,
            pltpu.SemaphoreType.DMA,
            pltpu.SemaphoreType.DMA,
            pltpu.SemaphoreType.DMA,
            pltpu.SemaphoreType.DMA,
        ],
        compiler_params=pltpu.CompilerParams(use_tc_tiling_on_sc=False),
    )
    def sc_kernel(x_hbm, src_hbm, dst_hbm, ones_hbm, zsum_hbm, zcnt_hbm,
                  out_sum, out_cnt,
                  sb0, sb1, sb2, dst_v, rb0, rb1, rb2, ones_v, sum_sh,
                  cnt_sh, si0, si1, si2, sg0, sg1, sg2, sem_c):
        c = lax.axis_index("core")
        s = lax.axis_index("subcore")
        w = c * NS + s
        ebase = w * chunks_pw * C

        sbuf = (sb0, sb1, sb2)
        bufs = (rb0, rb1, rb2)
        sem_i = (si0, si1, si2)
        sems = (sg0, sg1, sg2)

        pltpu.sync_copy(zsum_hbm.at[pl.ds(s * rpt, rpt)],
                        sum_sh.at[pl.ds(s * rpt, rpt)])
        pltpu.sync_copy(zcnt_hbm.at[pl.ds(s * rpt, rpt)],
                        cnt_sh.at[pl.ds(s * rpt, rpt)])
        pltpu.sync_copy(ones_hbm, ones_v)
        pltpu.sync_copy(dst_hbm.at[pl.ds(w * chunks_pw, chunks_pw)], dst_v)
        plsc.subcore_barrier()

        def load_src(k, m):
            pltpu.async_copy(src_hbm.at[pl.ds(ebase + k * C, C)],
                             sbuf[m], sem_i[m])

        def wait_src(m):
            pltpu.make_async_copy(src_hbm.at[pl.ds(ebase, C)],
                                  sbuf[m], sem_i[m]).wait()

        def start_gather(m):
            pltpu.async_copy(x_hbm.at[sbuf[m]], bufs[m], sems[m])

        def wait_gather(m):
            pltpu.make_async_copy(x_hbm.at[sbuf[m]], bufs[m],
                                  sems[m]).wait()

        def scatter(i, b):
            pltpu.async_copy(ones_v, cnt_sh.at[dst_v.at[i]], sem_c, add=True)
            pltpu.sync_copy(bufs[b], sum_sh.at[dst_v.at[i]], add=True)

        load_src(0, 0)
        load_src(1, 1)
        load_src(2, 2)
        wait_src(0)
        start_gather(0)
        wait_src(1)
        start_gather(1)

        def slot(k, b):
            wait_gather(b)
            load_src(k + 3, b)
            wait_src((b + 2) % 3)
            start_gather((b + 2) % 3)
            scatter(k, b)

        n_tri = (chunks_pw - 5) // 3

        @pl.loop(0, n_tri)
        def _(q):
            k = 3 * q
            slot(k, 0)
            slot(k + 1, 1)
            slot(k + 2, 2)

        for k in range(3 * n_tri, chunks_pw):
            b = k % 3
            wait_gather(b)
            if k + 3 < chunks_pw:
                load_src(k + 3, b)
            if k + 2 < chunks_pw:
                wait_src((b + 2) % 3)
                start_gather((b + 2) % 3)
            scatter(k, b)

        @pl.loop(0, chunks_pw)
        def _(i):
            pltpu.make_async_copy(ones_v, cnt_sh.at[dst_v.at[0]],
                                  sem_c).wait()

        plsc.subcore_barrier()
        base = c * N + s * rpt
        pltpu.sync_copy(sum_sh.at[pl.ds(s * rpt, rpt)],
                        out_sum.at[pl.ds(base, rpt)])
        pltpu.sync_copy(cnt_sh.at[pl.ds(s * rpt, rpt)],
                        out_cnt.at[pl.ds(base, rpt)])

    return sc_kernel(x, src1, dst2, ones, zsum, zcnt)


def _tc_head(N, D, H, x, psum, pcnt, w1l_t, b1l, w1r_t, w2_t, b2):
    R = 2000
    G = N // R

    def body(x_r, p0_r, p1_r, c0_r, c1_r, w1l_r, b1l_r, w1r_r, w2_r, b2_r,
             o_r):
        ssum = p0_r[...] + p1_r[...]
        cnt = c0_r[...][:, :1] + c1_r[...][:, :1]
        agg = ssum / jnp.maximum(cnt, 1.0)
        h = lax.dot_general(agg, w1l_r[...], (((1,), (0,)), ((), ())),
                            preferred_element_type=jnp.float32)
        h = h + lax.dot_general(x_r[...], w1r_r[...], (((1,), (0,)), ((), ())),
                                preferred_element_type=jnp.float32)
        h = jnp.maximum(h + b1l_r[...], 0.0)
        o = lax.dot_general(h, w2_r[...], (((1,), (0,)), ((), ())),
                            preferred_element_type=jnp.float32)
        o_r[...] = jnp.maximum(o + b2_r[...], 0.0)

    return pl.pallas_call(
        body,
        grid=(G,),
        in_specs=[
            pl.BlockSpec((R, D), lambda i: (i, 0)),
            pl.BlockSpec((R, D), lambda i: (i, 0)),
            pl.BlockSpec((R, D), lambda i: (i + G, 0)),
            pl.BlockSpec((R, LN), lambda i: (i, 0)),
            pl.BlockSpec((R, LN), lambda i: (i + G, 0)),
            pl.BlockSpec((D, D), lambda i: (0, 0)),
            pl.BlockSpec((1, D), lambda i: (0, 0)),
            pl.BlockSpec((D, D), lambda i: (0, 0)),
            pl.BlockSpec((D, H), lambda i: (0, 0)),
            pl.BlockSpec((1, H), lambda i: (0, 0)),
        ],
        out_specs=pl.BlockSpec((R, H), lambda i: (i, 0)),
        out_shape=jax.ShapeDtypeStruct((N, H), jnp.float32),
    )(x, psum, psum, pcnt, pcnt, w1l_t, b1l, w1r_t, w2_t, b2)


def kernel(x, edge_index, W1l, b1l, W1r, W2, b2):
    N, D = x.shape
    E = edge_index.shape[1]
    H = W2.shape[0]
    assert E % (NW * C) == 0 and N % NS == 0
    assert E // (NW * C) >= 3

    src1 = edge_index[0]
    dst2 = edge_index[1].reshape(E // C, C)
    ones = jnp.ones((C, LN), jnp.float32)
    zsum = jnp.zeros((N, D), jnp.float32)
    zcnt = jnp.zeros((N, LN), jnp.float32)

    psum, pcnt = _sc_aggregate(N, D, E, x, src1, dst2, ones, zsum, zcnt)
    return _tc_head(N, D, H, x, psum, pcnt, W1l.T, b1l.reshape(1, D),
                    W1r.T, W2.T, b2.reshape(1, H))
